# Initial kernel scaffold; baseline (speedup 1.0000x reference)
import jax
import jax.numpy as jnp
from jax.experimental import pallas as pl


def kernel(x, edge_index, eps, W1, b1, gamma1, beta1, W2, b2, gamma2, beta2):
    raise NotImplementedError("write your pallas kernel here")


# speedup vs baseline: 4.5532x; 4.5532x over previous
import functools

import jax
import jax.numpy as jnp
from jax import lax
from jax.experimental import pallas as pl
from jax.experimental.pallas import tpu as pltpu
from jax.experimental.pallas import tpu_sc as plsc

N = 10000
DI = 128
DO = 128

NC = 2
NS = 16
NW = NC * NS
CHUNK = 128

N_PAD = 10112
ROWS_PER_SUB = N_PAD // NS


def _sc_aggregate(x, src3, dst3, zeros):
    cpw = src3.shape[1]
    mesh = plsc.VectorSubcoreMesh(core_axis_name="c", subcore_axis_name="s")

    @functools.partial(
        pl.kernel,
        out_type=jax.ShapeDtypeStruct((NC, N_PAD, DI), jnp.float32),
        mesh=mesh,
        scratch_types=[
            pltpu.VMEM((cpw, CHUNK), jnp.int32),
            pltpu.VMEM((cpw, CHUNK), jnp.int32),
            pltpu.VMEM((CHUNK, DI), jnp.float32),
            pltpu.VMEM_SHARED((N_PAD, DI), jnp.float32),
            pltpu.SemaphoreType.DMA,
        ],
    )
    def k(x_hbm, src_hbm, dst_hbm, zeros_hbm, out_hbm,
          src_v, dst_v, rows_v, agg_sh, sem):
        cid = lax.axis_index("c")
        sid = lax.axis_index("s")
        wid = cid * NS + sid
        my_rows = pl.ds(sid * ROWS_PER_SUB, ROWS_PER_SUB)
        pltpu.sync_copy(zeros_hbm.at[my_rows], agg_sh.at[my_rows])
        pltpu.sync_copy(src_hbm.at[wid], src_v)
        pltpu.sync_copy(dst_hbm.at[wid], dst_v)
        plsc.subcore_barrier()

        def body(j, carry):
            pltpu.async_copy(x_hbm.at[src_v.at[j]], rows_v, sem).wait()
            pltpu.sync_copy(rows_v, agg_sh.at[dst_v.at[j]], add=True)
            return carry

        lax.fori_loop(0, cpw, body, 0, unroll=False)
        plsc.subcore_barrier()
        pltpu.sync_copy(agg_sh.at[my_rows], out_hbm.at[cid].at[my_rows])

    return k(x, src3, dst3, zeros)


def _tc_mlp(x, parts, eps, W1, b1, g1, be1, W2, b2, g2, be2):
    def body(x_ref, p_ref, eps_ref, W1_ref, b1_ref, g1_ref, be1_ref,
             W2_ref, b2_ref, g2_ref, be2_ref, o_ref):
        agg = p_ref[0, :N, :] + p_ref[1, :N, :]
        h = (1.0 + eps_ref[0]) * x_ref[...] + agg
        y = jnp.dot(h, W1_ref[...], preferred_element_type=jnp.float32) + b1_ref[...]
        mu = jnp.mean(y, axis=0, keepdims=True)
        yc = y - mu
        var = jnp.mean(yc * yc, axis=0, keepdims=True)
        y = g1_ref[...] * yc * lax.rsqrt(var + 1e-5) + be1_ref[...]
        y = jnp.maximum(y, 0.0)
        z = jnp.dot(y, W2_ref[...], preferred_element_type=jnp.float32) + b2_ref[...]
        mu2 = jnp.mean(z, axis=0, keepdims=True)
        zc = z - mu2
        var2 = jnp.mean(zc * zc, axis=0, keepdims=True)
        z = g2_ref[...] * zc * lax.rsqrt(var2 + 1e-5) + be2_ref[...]
        o_ref[...] = jnp.maximum(z, 0.0)

    return pl.pallas_call(
        body,
        out_shape=jax.ShapeDtypeStruct((N, DO), jnp.float32),
    )(x, parts, eps, W1, b1, g1, be1, W2, b2, g2, be2)


def kernel(x, edge_index, eps, W1, b1, gamma1, beta1, W2, b2, gamma2, beta2):
    dst = edge_index[0].astype(jnp.int32)
    src = edge_index[1].astype(jnp.int32)
    e = dst.shape[0]
    epw = -(-e // NW)
    cpw = -(-epw // CHUNK)
    e_pad = NW * cpw * CHUNK
    pad = e_pad - e
    src3 = jnp.concatenate([src, jnp.zeros((pad,), jnp.int32)]).reshape(NW, cpw, CHUNK)
    dst3 = jnp.concatenate([dst, jnp.full((pad,), N, jnp.int32)]).reshape(NW, cpw, CHUNK)
    zeros = jnp.zeros((N_PAD, DI), jnp.float32)
    parts = _sc_aggregate(x, src3, dst3, zeros)
    return _tc_mlp(x, parts, eps, W1, b1, gamma1, beta1, W2, b2, gamma2, beta2)
